# Initial kernel scaffold; baseline (speedup 1.0000x reference)
import jax
import jax.numpy as jnp
from jax.experimental import pallas as pl


def kernel(input_ids, table):
    raise NotImplementedError("write your pallas kernel here")


# speedup vs baseline: 1.4673x; 1.4673x over previous
import functools

import jax
import jax.numpy as jnp
from jax import lax
from jax.experimental import pallas as pl
from jax.experimental.pallas import tpu as pltpu
from jax.experimental.pallas import tpu_sc as plsc

_NUM_CORES = 2
_NUM_SUBCORES = 16
_NUM_WORKERS = _NUM_CORES * _NUM_SUBCORES


@functools.partial(jax.jit, static_argnames=("chunk",))
def _sc_gather(idx_flat, table, chunk=1280):
  n = idx_flat.shape[0]
  v, d = table.shape
  n_per_w = n // _NUM_WORKERS
  n_chunks = n_per_w // chunk
  assert n_per_w % chunk == 0

  mesh = plsc.VectorSubcoreMesh(
      core_axis_name="c", subcore_axis_name="s",
      num_cores=_NUM_CORES, num_subcores=_NUM_SUBCORES)

  @functools.partial(
      pl.kernel,
      mesh=mesh,
      compiler_params=pltpu.CompilerParams(use_tc_tiling_on_sc=False),
      out_type=jax.ShapeDtypeStruct((n, d), jnp.float32),
      scratch_types=[
          pltpu.VMEM((chunk,), jnp.int32),
          pltpu.VMEM((chunk, d), jnp.float32),
          pltpu.SemaphoreType.DMA,
      ],
  )
  def k(idx_hbm, table_hbm, out_hbm, idx_v, rows_v, sem):
    wid = lax.axis_index("s") * _NUM_CORES + lax.axis_index("c")
    base = wid * n_per_w

    def body(i, carry):
      off = base + i * chunk
      pltpu.sync_copy(idx_hbm.at[pl.ds(off, chunk)], idx_v)
      pltpu.async_copy(table_hbm.at[idx_v], rows_v, sem).wait()
      pltpu.sync_copy(rows_v, out_hbm.at[pl.ds(off, chunk)])
      return carry

    lax.fori_loop(0, n_chunks, body, 0)

  return k(idx_flat, table)


def kernel(input_ids, table):
  b, s = input_ids.shape
  idx_flat = input_ids.reshape(b * s).astype(jnp.int32)
  out = _sc_gather(idx_flat, table)
  return out.reshape(b, s, table.shape[1])


# speedup vs baseline: 1.4876x; 1.0138x over previous
import functools

import jax
import jax.numpy as jnp
from jax import lax
from jax.experimental import pallas as pl
from jax.experimental.pallas import tpu as pltpu
from jax.experimental.pallas import tpu_sc as plsc

_NUM_CORES = 2
_NUM_SUBCORES = 16
_NUM_WORKERS = _NUM_CORES * _NUM_SUBCORES


@functools.partial(jax.jit, static_argnames=("chunk",))
def _sc_gather(idx_flat, table, chunk=1280):
  n = idx_flat.shape[0]
  v, d = table.shape
  n_per_w = n // _NUM_WORKERS
  n_chunks = n_per_w // chunk
  assert n_per_w % chunk == 0

  mesh = plsc.VectorSubcoreMesh(
      core_axis_name="c", subcore_axis_name="s",
      num_cores=_NUM_CORES, num_subcores=_NUM_SUBCORES)

  @functools.partial(
      pl.kernel,
      mesh=mesh,
      compiler_params=pltpu.CompilerParams(use_tc_tiling_on_sc=False),
      out_type=jax.ShapeDtypeStruct((n, d), jnp.float32),
      scratch_types=[
          pltpu.VMEM((2, chunk), jnp.int32),
          pltpu.VMEM((2, chunk, d), jnp.float32),
          pltpu.SemaphoreType.DMA,
      ],
  )
  def k(idx_hbm, table_hbm, out_hbm, idx_v, rows_v, sem):
    wid = lax.axis_index("s") * _NUM_CORES + lax.axis_index("c")
    base = wid * n_per_w
    n_groups = n_chunks // 2

    def stage_and_fire(c, slot):
      pltpu.sync_copy(idx_hbm.at[pl.ds(base + c * chunk, chunk)],
                      idx_v.at[slot])
      pltpu.async_copy(table_hbm.at[idx_v.at[slot]], rows_v.at[slot], sem)

    def drain_and_flush(c, slot):
      pltpu.make_async_copy(table_hbm.at[idx_v.at[slot]], rows_v.at[slot],
                            sem).wait()
      pltpu.sync_copy(rows_v.at[slot], out_hbm.at[pl.ds(base + c * chunk,
                                                        chunk)])

    stage_and_fire(0, 0)

    def body(g, carry):
      stage_and_fire(2 * g + 1, 1)
      drain_and_flush(2 * g, 0)

      @pl.when(g < n_groups - 1)
      def _():
        stage_and_fire(2 * g + 2, 0)

      drain_and_flush(2 * g + 1, 1)
      return carry

    lax.fori_loop(0, n_groups, body, 0)

  return k(idx_flat, table)


def kernel(input_ids, table):
  b, s = input_ids.shape
  idx_flat = input_ids.reshape(b * s).astype(jnp.int32)
  out = _sc_gather(idx_flat, table)
  return out.reshape(b, s, table.shape[1])
